# baseline (speedup 1.0000x reference)
import jax
import jax.numpy as jnp
from jax.experimental import pallas as pl
from jax.experimental.pallas import tpu as pltpu

_N = 10000
_D = 128
_BM = 200
_MB = _N // _BM
_BR = 5000
_RB = _N // _BR
_BN_EPS = 1e-5


def _fused_kernel(adj_ref, x_ref, w_ref, g_ref, b_ref, y_ref,
                  acc_ref, cs_ref, css_ref):
    i = pl.program_id(0)

    @pl.when(i < _MB)
    def _compute():
        tmp = jnp.dot(adj_ref[...], x_ref[...],
                      preferred_element_type=jnp.float32)
        out = jnp.dot(tmp, w_ref[...], preferred_element_type=jnp.float32)
        acc_ref[pl.ds(i * _BM, _BM), :] = out
        s = jnp.sum(out, axis=0, keepdims=True)
        sq = jnp.sum(out * out, axis=0, keepdims=True)

        @pl.when(i == 0)
        def _():
            cs_ref[...] = s
            css_ref[...] = sq

        @pl.when(i != 0)
        def _():
            cs_ref[...] = cs_ref[...] + s
            css_ref[...] = css_ref[...] + sq

    @pl.when(i >= _MB)
    def _normalize():
        mean = cs_ref[...] * (1.0 / _N)
        var = css_ref[...] * (1.0 / _N) - mean * mean
        inv = jax.lax.rsqrt(var + _BN_EPS)
        out = acc_ref[pl.ds((i - _MB) * _BR, _BR), :]
        y_ref[...] = jnp.tanh((out - mean) * inv * g_ref[...] + b_ref[...])


def kernel(input, adj, W, bn_weight, bn_bias):
    g = bn_weight.reshape(1, _D)
    b = bn_bias.reshape(1, _D)
    y = pl.pallas_call(
        _fused_kernel,
        grid=(_MB + _RB,),
        in_specs=[
            pl.BlockSpec((_BM, _N), lambda i: (jnp.minimum(i, _MB - 1), 0)),
            pl.BlockSpec((_N, _D), lambda i: (0, 0)),
            pl.BlockSpec((_D, _D), lambda i: (0, 0)),
            pl.BlockSpec((1, _D), lambda i: (0, 0)),
            pl.BlockSpec((1, _D), lambda i: (0, 0)),
        ],
        out_specs=pl.BlockSpec((_BR, _D),
                               lambda i: (jnp.maximum(i - _MB, 0), 0)),
        out_shape=jax.ShapeDtypeStruct((_N, _D), jnp.float32),
        scratch_shapes=[
            pltpu.VMEM((_N, _D), jnp.float32),
            pltpu.VMEM((1, _D), jnp.float32),
            pltpu.VMEM((1, _D), jnp.float32),
        ],
    )(adj, input, W, g, b)
    return y


# speedup vs baseline: 1.0220x; 1.0220x over previous
import jax
import jax.numpy as jnp
from jax.experimental import pallas as pl
from jax.experimental.pallas import tpu as pltpu

_N = 10000
_D = 128
_BM = 400
_MB = _N // _BM
_BR = 10000
_RB = _N // _BR
_BN_EPS = 1e-5


def _fused_kernel(adj_ref, x_ref, w_ref, g_ref, b_ref, y_ref,
                  acc_ref, cs_ref, css_ref):
    i = pl.program_id(0)

    @pl.when(i < _MB)
    def _compute():
        tmp = jnp.dot(adj_ref[...], x_ref[...],
                      preferred_element_type=jnp.float32)
        out = jnp.dot(tmp, w_ref[...], preferred_element_type=jnp.float32)
        acc_ref[pl.ds(i * _BM, _BM), :] = out
        s = jnp.sum(out, axis=0, keepdims=True)
        sq = jnp.sum(out * out, axis=0, keepdims=True)

        @pl.when(i == 0)
        def _():
            cs_ref[...] = s
            css_ref[...] = sq

        @pl.when(i != 0)
        def _():
            cs_ref[...] = cs_ref[...] + s
            css_ref[...] = css_ref[...] + sq

    @pl.when(i >= _MB)
    def _normalize():
        mean = cs_ref[...] * (1.0 / _N)
        var = css_ref[...] * (1.0 / _N) - mean * mean
        inv = jax.lax.rsqrt(var + _BN_EPS)
        out = acc_ref[pl.ds((i - _MB) * _BR, _BR), :]
        y_ref[...] = jnp.tanh((out - mean) * inv * g_ref[...] + b_ref[...])


def kernel(input, adj, W, bn_weight, bn_bias):
    g = bn_weight.reshape(1, _D)
    b = bn_bias.reshape(1, _D)
    y = pl.pallas_call(
        _fused_kernel,
        grid=(_MB + _RB,),
        in_specs=[
            pl.BlockSpec((_BM, _N), lambda i: (jnp.minimum(i, _MB - 1), 0)),
            pl.BlockSpec((_N, _D), lambda i: (0, 0)),
            pl.BlockSpec((_D, _D), lambda i: (0, 0)),
            pl.BlockSpec((1, _D), lambda i: (0, 0)),
            pl.BlockSpec((1, _D), lambda i: (0, 0)),
        ],
        out_specs=pl.BlockSpec((_BR, _D),
                               lambda i: (jnp.maximum(i - _MB, 0), 0)),
        out_shape=jax.ShapeDtypeStruct((_N, _D), jnp.float32),
        scratch_shapes=[
            pltpu.VMEM((_N, _D), jnp.float32),
            pltpu.VMEM((1, _D), jnp.float32),
            pltpu.VMEM((1, _D), jnp.float32),
        ],
    )(adj, input, W, g, b)
    return y


# speedup vs baseline: 1.0259x; 1.0039x over previous
import jax
import jax.numpy as jnp
from jax.experimental import pallas as pl
from jax.experimental.pallas import tpu as pltpu

_N = 10000
_D = 128
_BM = 400
_MB = _N // _BM
_BR = 5000
_RB = _N // _BR
_BN_EPS = 1e-5


def _fused_kernel(adj_ref, x_ref, w_ref, g_ref, b_ref, y_ref,
                  acc_ref, cs_ref, css_ref):
    i = pl.program_id(0)

    @pl.when(i < _MB)
    def _compute():
        tmp = jnp.dot(adj_ref[...], x_ref[...],
                      preferred_element_type=jnp.float32)
        out = jnp.dot(tmp, w_ref[...], preferred_element_type=jnp.float32)
        acc_ref[pl.ds(i * _BM, _BM), :] = out
        s = jnp.sum(out, axis=0, keepdims=True)
        sq = jnp.sum(out * out, axis=0, keepdims=True)

        @pl.when(i == 0)
        def _():
            cs_ref[...] = s
            css_ref[...] = sq

        @pl.when(i != 0)
        def _():
            cs_ref[...] = cs_ref[...] + s
            css_ref[...] = css_ref[...] + sq

    @pl.when(i >= _MB)
    def _normalize():
        mean = cs_ref[...] * (1.0 / _N)
        var = css_ref[...] * (1.0 / _N) - mean * mean
        inv = jax.lax.rsqrt(var + _BN_EPS)
        out = acc_ref[pl.ds((i - _MB) * _BR, _BR), :]
        y_ref[...] = jnp.tanh((out - mean) * inv * g_ref[...] + b_ref[...])


def kernel(input, adj, W, bn_weight, bn_bias):
    g = bn_weight.reshape(1, _D)
    b = bn_bias.reshape(1, _D)
    y = pl.pallas_call(
        _fused_kernel,
        grid=(_MB + _RB,),
        in_specs=[
            pl.BlockSpec((_BM, _N), lambda i: (jnp.minimum(i, _MB - 1), 0)),
            pl.BlockSpec((_N, _D), lambda i: (0, 0)),
            pl.BlockSpec((_D, _D), lambda i: (0, 0)),
            pl.BlockSpec((1, _D), lambda i: (0, 0)),
            pl.BlockSpec((1, _D), lambda i: (0, 0)),
        ],
        out_specs=pl.BlockSpec((_BR, _D),
                               lambda i: (jnp.maximum(i - _MB, 0), 0)),
        out_shape=jax.ShapeDtypeStruct((_N, _D), jnp.float32),
        scratch_shapes=[
            pltpu.VMEM((_N, _D), jnp.float32),
            pltpu.VMEM((1, _D), jnp.float32),
            pltpu.VMEM((1, _D), jnp.float32),
        ],
    )(adj, input, W, g, b)
    return y
